# speedup vs baseline: 2.0367x; 2.0367x over previous
import functools

import jax
import jax.numpy as jnp
from jax import lax
from jax.experimental import pallas as pl
from jax.experimental.pallas import tpu as pltpu
from jax.experimental.pallas import tpu_sc as plsc

_NUM_EXPERT = 8
_IN = 1024
_OUT = 1024
_TOKENS = 2048
_BT = 256
_NB = _TOKENS // _BT
_NW = _NB + _NUM_EXPERT - 1


def _sc_permute(src, idx, scatter):
    rows, feat = src.shape
    mesh = plsc.VectorSubcoreMesh(core_axis_name="c", subcore_axis_name="s")
    nworker = mesh.num_cores * mesh.num_subcores
    per_w = rows // nworker

    @functools.partial(
        pl.kernel,
        mesh=mesh,
        out_type=jax.ShapeDtypeStruct((rows, feat), src.dtype),
        scratch_types=[
            pltpu.VMEM((per_w,), jnp.int32),
            pltpu.VMEM((per_w, feat), src.dtype),
            pltpu.SemaphoreType.DMA,
        ],
    )
    def k(src_hbm, idx_hbm, out_hbm, idx_v, rows_v, sem):
        wid = lax.axis_index("s") * mesh.num_cores + lax.axis_index("c")
        base = wid * per_w
        pltpu.sync_copy(idx_hbm.at[pl.ds(base, per_w)], idx_v)
        if scatter:
            pltpu.sync_copy(src_hbm.at[pl.ds(base, per_w)], rows_v)
            pltpu.async_copy(rows_v, out_hbm.at[idx_v], sem).wait()
        else:
            pltpu.async_copy(src_hbm.at[idx_v], rows_v, sem).wait()
            pltpu.sync_copy(rows_v, out_hbm.at[pl.ds(base, per_w)])

    return k(src, idx)


def _mm_body(meta_ref, x_ref, w_ref, g_ref, o_ref):
    w = pl.program_id(0)
    e_mask = meta_ref[2, w]
    first = meta_ref[3, w]
    xm = jnp.where(g_ref[...] == e_mask, x_ref[...], 0.0)
    part = lax.dot_general(
        xm, w_ref[0],
        dimension_numbers=(((1,), (1,)), ((), ())),
        preferred_element_type=jnp.float32,
    )

    @pl.when(first == 1)
    def _():
        o_ref[...] = part

    @pl.when(first == 0)
    def _():
        o_ref[...] += part


def _grouped_matmul(x_sorted, weight, sorted_gate, meta):
    grid_spec = pltpu.PrefetchScalarGridSpec(
        num_scalar_prefetch=1,
        grid=(_NW,),
        in_specs=[
            pl.BlockSpec((_BT, _IN), lambda w, m: (m[0, w], 0)),
            pl.BlockSpec((1, _OUT, _IN), lambda w, m: (m[1, w], 0, 0)),
            pl.BlockSpec((_BT, 1), lambda w, m: (m[0, w], 0)),
        ],
        out_specs=pl.BlockSpec((_BT, _OUT), lambda w, m: (m[0, w], 0)),
    )
    return pl.pallas_call(
        _mm_body,
        grid_spec=grid_spec,
        out_shape=jax.ShapeDtypeStruct((_TOKENS, _OUT), jnp.float32),
        compiler_params=pltpu.CompilerParams(
            dimension_semantics=("arbitrary",),
        ),
    )(meta, x_sorted, weight, sorted_gate.reshape(_TOKENS, 1))


def _routing(gate):
    g = gate.astype(jnp.int32)
    t = g.shape[0]
    eids = jnp.arange(_NUM_EXPERT, dtype=jnp.int32)
    oh = (g[:, None] == eids[None, :]).astype(jnp.int32)
    counts = jnp.sum(oh, axis=0)
    off = jnp.cumsum(counts) - counts
    pos = jnp.sum(oh * (jnp.cumsum(oh, axis=0) - 1), axis=1)
    dest = (pos + jnp.sum(oh * off[None, :], axis=1)).astype(jnp.int32)
    slot = jnp.arange(t, dtype=jnp.int32)
    sorted_gate = (jnp.sum((slot[:, None] >= off[None, :]).astype(jnp.int32),
                           axis=1) - 1).astype(jnp.int32)
    sgb = sorted_gate.reshape(_NB, _BT)
    e_lo, e_hi = sgb[:, 0], sgb[:, -1]
    nitem = e_hi - e_lo + 1
    starts = jnp.cumsum(nitem) - nitem
    total = jnp.sum(nitem)
    warr = jnp.arange(_NW, dtype=jnp.int32)
    b_of = jnp.sum((warr[:, None] >= starts[None, :]).astype(jnp.int32),
                   axis=1) - 1
    e_w = e_lo[b_of] + warr - starts[b_of]
    e_mask = jnp.where(warr < total, e_w, -1)
    e_load = jnp.clip(e_w, 0, _NUM_EXPERT - 1)
    firsts = (warr == starts[b_of]).astype(jnp.int32)
    meta = jnp.stack([b_of, e_load, e_mask, firsts]).astype(jnp.int32)
    return dest, sorted_gate, meta


def kernel(inp, gate, weight):
    dest, sorted_gate, meta = _routing(gate)
    x_sorted = _sc_permute(inp, dest, scatter=True)
    y_sorted = _grouped_matmul(x_sorted, weight, sorted_gate, meta)
    return _sc_permute(y_sorted, dest, scatter=False)
